# Initial kernel scaffold; baseline (speedup 1.0000x reference)
import jax
import jax.numpy as jnp
from jax.experimental import pallas as pl


def kernel(inputs, unq_inv, W, gamma, beta):
    raise NotImplementedError("write your pallas kernel here")


# speedup vs baseline: 1.2379x; 1.2379x over previous
import functools

import jax
import jax.numpy as jnp
from jax import lax
from jax.experimental import pallas as pl
from jax.experimental.pallas import tpu as pltpu
from jax.experimental.pallas import tpu_sc as plsc

N = 320000
IN_CH = 128
UNITS = 64
NUM_SEG = 10000
EPS = 1e-3

TILE = 512
GRID = N // TILE
NW = 32
CHUNK = 512
CHUNK2 = 256
SEG_PW = (NUM_SEG + NW - 1) // NW + 4
SLAB = SEG_PW + 3
NSEG_PAD = NUM_SEG + 16
N_PAD = N + CHUNK


def _mm_stats_body(in_ref, ids_ref, w_ref, x_ref, stats_ref, counts_ref):
    i = pl.program_id(0)

    @pl.when(i == 0)
    def _():
        stats_ref[...] = jnp.zeros_like(stats_ref)
        counts_ref[...] = jnp.zeros_like(counts_ref)

    x = lax.dot_general(in_ref[...], w_ref[...],
                        (((1,), (1,)), ((), ())),
                        preferred_element_type=jnp.float32)
    x_ref[...] = x
    s = jnp.sum(x, axis=0, keepdims=True)
    s2 = jnp.sum(x * x, axis=0, keepdims=True)
    stats_ref[0:1, :] += s
    stats_ref[1:2, :] += s2

    ids = ids_ref[0, 0, :].astype(jnp.int32)
    wix = lax.broadcasted_iota(jnp.int32, (64, 1), 0)
    thr = ((wix + 1) * NUM_SEG) // NW
    cmp = (ids[None, :] < thr).astype(jnp.int32)
    counts_ref[:, 0:1] += jnp.sum(cmp, axis=1, keepdims=True)


def _run_mm_stats(inputs, ids3, W):
    return pl.pallas_call(
        _mm_stats_body,
        grid=(GRID,),
        in_specs=[
            pl.BlockSpec((TILE, IN_CH), lambda i: (i, 0)),
            pl.BlockSpec((1, 1, TILE), lambda i: (i, 0, 0)),
            pl.BlockSpec((UNITS, IN_CH), lambda i: (0, 0)),
        ],
        out_specs=[
            pl.BlockSpec((TILE, UNITS), lambda i: (i, 0)),
            pl.BlockSpec((8, UNITS), lambda i: (0, 0)),
            pl.BlockSpec((64, 128), lambda i: (0, 0)),
        ],
        out_shape=[
            jax.ShapeDtypeStruct((N_PAD, UNITS), jnp.float32),
            jax.ShapeDtypeStruct((8, UNITS), jnp.float32),
            jax.ShapeDtypeStruct((64, 128), jnp.int32),
        ],
    )(inputs, ids3, W)


def _bn_swish_body(x_ref, stats_ref, g_ref, b_ref, y_ref):
    mean = stats_ref[0:1, :] / N
    ex2 = stats_ref[1:2, :] / N
    var = ex2 - mean * mean
    inv = lax.rsqrt(var + EPS)
    a = g_ref[...] * inv
    b = b_ref[...] - mean * a
    t = x_ref[...] * a + b
    y_ref[...] = t * (1.0 / (1.0 + jnp.exp(-t)))


def _run_bn_swish(x, stats, gamma, beta):
    return pl.pallas_call(
        _bn_swish_body,
        grid=(GRID,),
        in_specs=[
            pl.BlockSpec((TILE, UNITS), lambda i: (i, 0)),
            pl.BlockSpec((8, UNITS), lambda i: (0, 0)),
            pl.BlockSpec((1, UNITS), lambda i: (0, 0)),
            pl.BlockSpec((1, UNITS), lambda i: (0, 0)),
        ],
        out_specs=pl.BlockSpec((TILE, UNITS), lambda i: (i, 0)),
        out_shape=jax.ShapeDtypeStruct((N_PAD, UNITS), jnp.float32),
    )(x, stats, gamma, beta)


def _tail_copy(src, dst_hbm, base, cnt):
    off = jnp.int32(0)
    sz = src.shape[0] // 2
    while sz >= 8:
        here = (cnt & sz) != 0

        @pl.when(here)
        def _(off=off, sz=sz):
            o8 = pl.multiple_of(base + off, 8)
            pltpu.sync_copy(src.at[pl.ds(off, sz), :],
                            dst_hbm.at[pl.ds(o8, sz), :])

        off = off + jnp.where(here, sz, 0)
        sz //= 2


def _sc1_body(y_hbm, ids_hbm, bounds_hbm, feat_hbm,
              bvbuf, idvbuf, ybuf, feat):
    wid = lax.axis_index("c") * 16 + lax.axis_index("s")
    pltpu.sync_copy(bounds_hbm, bvbuf.at[pl.ds(0, 40)])
    lo = bvbuf[pl.ds(wid, 16)][0]
    hi = bvbuf[pl.ds(wid + 1, 16)][0]
    n = hi - lo
    segbase = (wid * NUM_SEG) // NW
    nsegs = ((wid + 1) * NUM_SEG) // NW - segbase
    nch = (n + CHUNK - 1) // CHUNK

    minf = jnp.full((16,), -jnp.inf, jnp.float32)

    @pl.when(n > 0)
    def _():
        def chunk1(ci, carry):
            m0, m1, m2, m3, prev = carry
            base = lo + ci * CHUNK
            cnt = jnp.minimum(CHUNK, n - ci * CHUNK)
            base_al = pl.multiple_of((base // 8) * 8, 8)
            o = base - base_al
            pltpu.sync_copy(y_hbm.at[pl.ds(base_al, CHUNK + 8), :], ybuf)
            pltpu.sync_copy(ids_hbm.at[pl.ds(base_al, CHUNK + 8)],
                            idvbuf.at[pl.ds(0, CHUNK + 8)])

            def row(r, c):
                m0, m1, m2, m3, prev = c
                q = o + r
                sid = idvbuf[pl.ds(q, 16)][0]
                change = sid != prev

                @pl.when(change)
                def _():
                    foff = (prev - segbase) * 64
                    feat[pl.ds(foff, 16)] = m0
                    feat[pl.ds(foff + 16, 16)] = m1
                    feat[pl.ds(foff + 32, 16)] = m2
                    feat[pl.ds(foff + 48, 16)] = m3

                y0 = ybuf[q, pl.ds(0, 16)]
                y1 = ybuf[q, pl.ds(16, 16)]
                y2 = ybuf[q, pl.ds(32, 16)]
                y3 = ybuf[q, pl.ds(48, 16)]
                m0 = jnp.where(change, y0, jnp.maximum(m0, y0))
                m1 = jnp.where(change, y1, jnp.maximum(m1, y1))
                m2 = jnp.where(change, y2, jnp.maximum(m2, y2))
                m3 = jnp.where(change, y3, jnp.maximum(m3, y3))
                return m0, m1, m2, m3, sid

            return lax.fori_loop(0, cnt, row, (m0, m1, m2, m3, prev))

        init = (minf, minf, minf, minf, segbase.astype(jnp.int32))
        m0, m1, m2, m3, prev = lax.fori_loop(0, nch, chunk1, init)
        foff = (prev - segbase) * 64
        feat[pl.ds(foff, 16)] = m0
        feat[pl.ds(foff + 16, 16)] = m1
        feat[pl.ds(foff + 32, 16)] = m2
        feat[pl.ds(foff + 48, 16)] = m3

        pltpu.sync_copy(feat.at[pl.ds(0, 312 * 64)],
                        feat_hbm.at[pl.ds(segbase * 64, 312 * 64)])

        @pl.when(nsegs > 312)
        def _():
            pltpu.sync_copy(
                feat.at[pl.ds(312 * 64, 64)],
                feat_hbm.at[pl.ds((segbase + 312) * 64, 64)])


def _sc2_body(y_hbm, ids_hbm, wal_hbm, feat_hbm, out_hbm,
              bvbuf, idvbuf, ybuf, slab, fbuf, obuf):
    wid = lax.axis_index("c") * 16 + lax.axis_index("s")
    pltpu.sync_copy(wal_hbm, bvbuf.at[pl.ds(0, 40)])
    lo = bvbuf[pl.ds(wid, 16)][0]
    hi = bvbuf[pl.ds(wid + 1, 16)][0]
    n = hi - lo
    segbase = (wid * NUM_SEG) // NW
    nch = (n + CHUNK2 - 1) // CHUNK2

    pltpu.sync_copy(feat_hbm.at[pl.ds(segbase * 64, SLAB * 64)], slab)

    @pl.when(n > 0)
    def _():
        def chunk2(ci, _carry):
            base = pl.multiple_of(lo + ci * CHUNK2, 8)
            cnt = jnp.minimum(CHUNK2, n - ci * CHUNK2)
            pltpu.sync_copy(y_hbm.at[pl.ds(base, CHUNK2), :], ybuf)
            pltpu.sync_copy(ids_hbm.at[pl.ds(base, CHUNK2 + 8)],
                            idvbuf.at[pl.ds(0, CHUNK2 + 8)])

            def row(r, _):
                sid = idvbuf[pl.ds(r, 16)][0]
                off = sid - segbase
                obuf[r, pl.ds(0, 16)] = ybuf[r, pl.ds(0, 16)]
                obuf[r, pl.ds(16, 16)] = ybuf[r, pl.ds(16, 16)]
                obuf[r, pl.ds(32, 16)] = ybuf[r, pl.ds(32, 16)]
                obuf[r, pl.ds(48, 16)] = ybuf[r, pl.ds(48, 16)]

                @pl.when(off >= 0)
                def _():
                    foff = off * 64
                    obuf[r, pl.ds(64, 16)] = slab[pl.ds(foff, 16)]
                    obuf[r, pl.ds(80, 16)] = slab[pl.ds(foff + 16, 16)]
                    obuf[r, pl.ds(96, 16)] = slab[pl.ds(foff + 32, 16)]
                    obuf[r, pl.ds(112, 16)] = slab[pl.ds(foff + 48, 16)]

                @pl.when(off < 0)
                def _():
                    pltpu.sync_copy(feat_hbm.at[pl.ds(sid * 64, 64)], fbuf)
                    obuf[r, pl.ds(64, 16)] = fbuf[pl.ds(0, 16)]
                    obuf[r, pl.ds(80, 16)] = fbuf[pl.ds(16, 16)]
                    obuf[r, pl.ds(96, 16)] = fbuf[pl.ds(32, 16)]
                    obuf[r, pl.ds(112, 16)] = fbuf[pl.ds(48, 16)]

                return 0

            lax.fori_loop(0, cnt, row, 0)

            @pl.when(cnt == CHUNK2)
            def _():
                pltpu.sync_copy(obuf, out_hbm.at[pl.ds(base, CHUNK2), :])

            @pl.when(cnt < CHUNK2)
            def _():
                _tail_copy(obuf, out_hbm, base, cnt)

            return 0

        lax.fori_loop(0, nch, chunk2, 0)


def _run_sc(y, ids_pad, bounds, walign):
    mesh = plsc.VectorSubcoreMesh(core_axis_name="c", subcore_axis_name="s")
    feat = pl.kernel(
        _sc1_body,
        out_type=jax.ShapeDtypeStruct((NSEG_PAD * 64,), jnp.float32),
        mesh=mesh,
        scratch_types=[
            pltpu.VMEM((56,), jnp.int32),
            pltpu.VMEM((CHUNK + 24,), jnp.int32),
            pltpu.VMEM((CHUNK + 8, UNITS), jnp.float32),
            pltpu.VMEM((SEG_PW * UNITS,), jnp.float32),
        ],
    )(y, ids_pad, bounds)
    out = pl.kernel(
        _sc2_body,
        out_type=jax.ShapeDtypeStruct((N, 128), jnp.float32),
        mesh=mesh,
        scratch_types=[
            pltpu.VMEM((56,), jnp.int32),
            pltpu.VMEM((CHUNK2 + 24,), jnp.int32),
            pltpu.VMEM((CHUNK2, UNITS), jnp.float32),
            pltpu.VMEM((SLAB * UNITS,), jnp.float32),
            pltpu.VMEM((UNITS,), jnp.float32),
            pltpu.VMEM((CHUNK2, 128), jnp.float32),
        ],
    )(y, ids_pad, walign, feat)
    return out


def kernel(inputs, unq_inv, W, gamma, beta):
    ids = unq_inv.astype(jnp.int32)
    ids3 = ids.reshape(GRID, 1, TILE)
    x, stats, counts = _run_mm_stats(inputs, ids3, W)
    y = _run_bn_swish(x, stats, gamma.reshape(1, UNITS), beta.reshape(1, UNITS))
    c = counts[:NW, 0]
    zero1 = jnp.zeros((1,), jnp.int32)
    pad7 = jnp.zeros((40 - NW - 1,), jnp.int32)
    bounds = jnp.concatenate([zero1, c, pad7])
    wal = jnp.concatenate(
        [zero1, (c[: NW - 1] // 8) * 8, jnp.full((1,), N, jnp.int32), pad7])
    ids_pad = jnp.pad(ids, (0, CHUNK + 8))
    return _run_sc(y, ids_pad, bounds, wal)


# speedup vs baseline: 2.7023x; 2.1830x over previous
import dataclasses
import functools

import jax
import jax.numpy as jnp
from jax import lax
from jax.experimental import pallas as pl
from jax.experimental.pallas import tpu as pltpu
from jax.experimental.pallas import tpu_sc as plsc

N = 320000
IN_CH = 128
UNITS = 64
NUM_SEG = 10000
EPS = 1e-3

TILE = 1280
GRID = N // TILE
NW = 32
CHUNK = 256
CHUNK2 = 96
SEG_PW = (NUM_SEG + NW - 1) // NW + 4
SLAB = SEG_PW + 3
NSEG_PAD = NUM_SEG + 16
N_PAD = N + CHUNK


def _mm_stats_body(in_ref, ids_ref, w_ref, x_ref, stats_ref, counts_ref,
                   acc_ref):
    i = pl.program_id(0)

    @pl.when(i == 0)
    def _():
        stats_ref[...] = jnp.zeros_like(stats_ref)
        acc_ref[...] = jnp.zeros_like(acc_ref)

    x = lax.dot_general(in_ref[...], w_ref[...],
                        (((1,), (1,)), ((), ())),
                        preferred_element_type=jnp.float32)
    x_ref[...] = x
    s = jnp.sum(x, axis=0, keepdims=True)
    s2 = jnp.sum(x * x, axis=0, keepdims=True)
    stats_ref[0:1, :] += s
    stats_ref[1:2, :] += s2

    ids = ids_ref[0, 0, :].astype(jnp.int32)
    wix = lax.broadcasted_iota(jnp.int32, (64, 1), 0)
    thr = ((wix + 1) * NUM_SEG) // NW
    acc_ref[...] += (ids[None, :] < thr).astype(jnp.int32)

    @pl.when(i == GRID - 1)
    def _():
        counts_ref[...] = jnp.sum(acc_ref[...], axis=1, keepdims=True)


def _run_mm_stats(inputs, ids3, W):
    return pl.pallas_call(
        _mm_stats_body,
        grid=(GRID,),
        in_specs=[
            pl.BlockSpec((TILE, IN_CH), lambda i: (i, 0)),
            pl.BlockSpec((1, 1, TILE), lambda i: (i, 0, 0)),
            pl.BlockSpec((UNITS, IN_CH), lambda i: (0, 0)),
        ],
        out_specs=[
            pl.BlockSpec((TILE, UNITS), lambda i: (i, 0)),
            pl.BlockSpec((8, UNITS), lambda i: (0, 0)),
            pl.BlockSpec((64, 1), lambda i: (0, 0)),
        ],
        out_shape=[
            jax.ShapeDtypeStruct((N_PAD, UNITS), jnp.float32),
            jax.ShapeDtypeStruct((8, UNITS), jnp.float32),
            jax.ShapeDtypeStruct((64, 1), jnp.int32),
        ],
        scratch_shapes=[pltpu.VMEM((64, TILE), jnp.int32)],
    )(inputs, ids3, W)


def _bn_swish_body(x_ref, stats_ref, g_ref, b_ref, y_ref):
    mean = stats_ref[0:1, :] / N
    ex2 = stats_ref[1:2, :] / N
    var = ex2 - mean * mean
    inv = lax.rsqrt(var + EPS)
    a = g_ref[...] * inv
    b = b_ref[...] - mean * a
    t = x_ref[...] * a + b
    y_ref[...] = t * (1.0 / (1.0 + jnp.exp(-t)))


def _run_bn_swish(x, stats, gamma, beta):
    return pl.pallas_call(
        _bn_swish_body,
        grid=(GRID,),
        in_specs=[
            pl.BlockSpec((TILE, UNITS), lambda i: (i, 0)),
            pl.BlockSpec((8, UNITS), lambda i: (0, 0)),
            pl.BlockSpec((1, UNITS), lambda i: (0, 0)),
            pl.BlockSpec((1, UNITS), lambda i: (0, 0)),
        ],
        out_specs=pl.BlockSpec((TILE, UNITS), lambda i: (i, 0)),
        out_shape=jax.ShapeDtypeStruct((N_PAD, UNITS), jnp.float32),
    )(x, stats, gamma, beta)


def _tail_start(src, dst_hbm, base, cnt, sem):
    off = jnp.int32(0)
    sz = 1 << (src.shape[0].bit_length() - 1)
    while sz >= 8:
        here = (cnt & sz) != 0

        @pl.when(here)
        def _(off=off, sz=sz):
            o8 = pl.multiple_of(base + off, 8)
            pltpu.async_copy(src.at[pl.ds(off, sz), :],
                             dst_hbm.at[pl.ds(o8, sz), :], sem)

        off = off + jnp.where(here, sz, 0)
        sz //= 2


def _tail_wait(src, dst_hbm, base, cnt, sem):
    off = jnp.int32(0)
    sz = 1 << (src.shape[0].bit_length() - 1)
    while sz >= 8:
        here = (cnt & sz) != 0

        @pl.when(here)
        def _(off=off, sz=sz):
            o8 = pl.multiple_of(base + off, 8)
            pltpu.make_async_copy(src.at[pl.ds(off, sz), :],
                                  dst_hbm.at[pl.ds(o8, sz), :], sem).wait()

        off = off + jnp.where(here, sz, 0)
        sz //= 2


def _find_run_end(idv, sid, q, qend):
    sidv = jnp.full((16,), sid, jnp.int32)

    def cond(p):
        in_range = p < qend
        same = plsc.all_reduce_ffs(idv[pl.ds(p, 16)] != sidv)[0] >= 16
        return jnp.logical_and(in_range, same)

    p = lax.while_loop(cond, lambda p: p + 16, q)
    f = plsc.all_reduce_ffs(idv[pl.ds(p, 16)] != sidv)[0]
    return jnp.maximum(jnp.minimum(p + f, qend), q + 1)


def _sc1_body(y_hbm, ids_hbm, bounds_hbm, feat_hbm,
              bvbuf, idv0, idv1, ybuf0, ybuf1, feat,
              sem_y0, sem_y1, sem_i0, sem_i1):
    wid = lax.axis_index("c") * 16 + lax.axis_index("s")
    pltpu.sync_copy(bounds_hbm, bvbuf.at[pl.ds(0, 40)])
    lo = bvbuf[pl.ds(wid, 16)][0]
    hi = bvbuf[pl.ds(wid + 1, 16)][0]
    n = hi - lo
    segbase = (wid * NUM_SEG) // NW
    nsegs = ((wid + 1) * NUM_SEG) // NW - segbase
    nch = (n + CHUNK - 1) // CHUNK

    ybufs = (ybuf0, ybuf1)
    idvs = (idv0, idv1)
    sems = ((sem_y0, sem_i0), (sem_y1, sem_i1))

    def issue(ci, b):
        base = lo + ci * CHUNK
        base_al = pl.multiple_of((base // 8) * 8, 8)
        pltpu.async_copy(y_hbm.at[pl.ds(base_al, CHUNK + 8), :],
                         ybufs[b], sems[b][0])
        pltpu.async_copy(ids_hbm.at[pl.ds(base_al, CHUNK + 8)],
                         idvs[b].at[pl.ds(0, CHUNK + 8)], sems[b][1])

    def wait(ci, b):
        base = lo + ci * CHUNK
        base_al = pl.multiple_of((base // 8) * 8, 8)
        pltpu.make_async_copy(y_hbm.at[pl.ds(base_al, CHUNK + 8), :],
                              ybufs[b], sems[b][0]).wait()
        pltpu.make_async_copy(ids_hbm.at[pl.ds(base_al, CHUNK + 8)],
                              idvs[b].at[pl.ds(0, CHUNK + 8)],
                              sems[b][1]).wait()

    minf = jnp.full((16,), -jnp.inf, jnp.float32)

    @pl.when(n > 0)
    def _():
        issue(0, 0)

        @pl.when(nch > 1)
        def _():
            issue(1, 1)

        def process(ci, b, carry):
            ybuf = ybufs[b]
            idv = idvs[b]
            wait(ci, b)
            base = lo + ci * CHUNK
            cnt = jnp.minimum(CHUNK, n - ci * CHUNK)
            o = base - (base // 8) * 8

            def run_body(c):
                r, m0, m1, m2, m3, prev = c
                q = o + r
                sid = idv[pl.ds(q, 16)][0]
                change = sid != prev

                @pl.when(change)
                def _():
                    foff = (prev - segbase) * 64
                    feat[pl.ds(foff, 16)] = m0
                    feat[pl.ds(foff + 16, 16)] = m1
                    feat[pl.ds(foff + 32, 16)] = m2
                    feat[pl.ds(foff + 48, 16)] = m3

                e = _find_run_end(idv, sid, q, o + cnt) - o

                y0 = ybuf[q, pl.ds(0, 16)]
                y1 = ybuf[q, pl.ds(16, 16)]
                y2 = ybuf[q, pl.ds(32, 16)]
                y3 = ybuf[q, pl.ds(48, 16)]
                m0 = jnp.where(change, y0, jnp.maximum(m0, y0))
                m1 = jnp.where(change, y1, jnp.maximum(m1, y1))
                m2 = jnp.where(change, y2, jnp.maximum(m2, y2))
                m3 = jnp.where(change, y3, jnp.maximum(m3, y3))

                def maxrow(rr, mm):
                    m0, m1, m2, m3 = mm
                    qq = o + rr
                    m0 = jnp.maximum(m0, ybuf[qq, pl.ds(0, 16)])
                    m1 = jnp.maximum(m1, ybuf[qq, pl.ds(16, 16)])
                    m2 = jnp.maximum(m2, ybuf[qq, pl.ds(32, 16)])
                    m3 = jnp.maximum(m3, ybuf[qq, pl.ds(48, 16)])
                    return m0, m1, m2, m3

                m0, m1, m2, m3 = lax.fori_loop(r + 1, e, maxrow,
                                               (m0, m1, m2, m3))
                return e, m0, m1, m2, m3, sid

            r, m0, m1, m2, m3, prev = lax.while_loop(
                lambda c: c[0] < cnt, run_body, carry)
            return jnp.int32(0), m0, m1, m2, m3, prev

        def outer(cj, carry):
            c = carry
            for b in range(2):
                ci = cj * 2 + b

                def do(c=c, ci=ci, b=b):
                    c2 = process(ci, b, c)

                    @pl.when(ci + 2 < nch)
                    def _():
                        issue(ci + 2, b)

                    return c2

                c = lax.cond(ci < nch, do, lambda c=c: c)
            return c

        init = (jnp.int32(0), minf, minf, minf, minf,
                segbase.astype(jnp.int32))
        fin = lax.fori_loop(0, (nch + 1) // 2, outer, init)
        _, m0, m1, m2, m3, prev = fin
        foff = (prev - segbase) * 64
        feat[pl.ds(foff, 16)] = m0
        feat[pl.ds(foff + 16, 16)] = m1
        feat[pl.ds(foff + 32, 16)] = m2
        feat[pl.ds(foff + 48, 16)] = m3

        pltpu.sync_copy(feat.at[pl.ds(0, 312 * 64)],
                        feat_hbm.at[pl.ds(segbase * 64, 312 * 64)])

        @pl.when(nsegs > 312)
        def _():
            pltpu.sync_copy(
                feat.at[pl.ds(312 * 64, 64)],
                feat_hbm.at[pl.ds((segbase + 312) * 64, 64)])


def _sc2_body(y_hbm, ids_hbm, wal_hbm, feat_hbm, out_hbm,
              bvbuf, idv0, idv1, ybuf0, ybuf1, slab, fbuf, obuf0, obuf1,
              sem_y0, sem_y1, sem_i0, sem_i1, sem_o0, sem_o1):
    wid = lax.axis_index("c") * 16 + lax.axis_index("s")
    pltpu.sync_copy(wal_hbm, bvbuf.at[pl.ds(0, 40)])
    lo = bvbuf[pl.ds(wid, 16)][0]
    hi = bvbuf[pl.ds(wid + 1, 16)][0]
    n = hi - lo
    segbase = (wid * NUM_SEG) // NW
    nch = (n + CHUNK2 - 1) // CHUNK2

    pltpu.sync_copy(feat_hbm.at[pl.ds(segbase * 64, SLAB * 64)], slab)

    ybufs = (ybuf0, ybuf1)
    idvs = (idv0, idv1)
    obufs = (obuf0, obuf1)
    isems = ((sem_y0, sem_i0), (sem_y1, sem_i1))
    osems = (sem_o0, sem_o1)

    def issue(ci, b):
        base = pl.multiple_of(lo + ci * CHUNK2, 8)
        pltpu.async_copy(y_hbm.at[pl.ds(base, CHUNK2), :],
                         ybufs[b], isems[b][0])
        pltpu.async_copy(ids_hbm.at[pl.ds(base, CHUNK2 + 8)],
                         idvs[b].at[pl.ds(0, CHUNK2 + 8)], isems[b][1])

    def wait_in(ci, b):
        base = pl.multiple_of(lo + ci * CHUNK2, 8)
        pltpu.make_async_copy(y_hbm.at[pl.ds(base, CHUNK2), :],
                              ybufs[b], isems[b][0]).wait()
        pltpu.make_async_copy(ids_hbm.at[pl.ds(base, CHUNK2 + 8)],
                              idvs[b].at[pl.ds(0, CHUNK2 + 8)],
                              isems[b][1]).wait()

    @pl.when(n > 0)
    def _():
        issue(0, 0)

        @pl.when(nch > 1)
        def _():
            issue(1, 1)

        def process(ci, b):
            ybuf = ybufs[b]
            idv = idvs[b]
            obuf = obufs[b]
            wait_in(ci, b)
            base = pl.multiple_of(lo + ci * CHUNK2, 8)
            cnt = jnp.minimum(CHUNK2, n - ci * CHUNK2)

            @pl.when(ci >= 2)
            def _():
                pb = pl.multiple_of(lo + (ci - 2) * CHUNK2, 8)
                pcnt = jnp.minimum(CHUNK2, n - (ci - 2) * CHUNK2)
                _tail_wait(obuf, out_hbm, pb, pcnt, osems[b])

            def run_body(c):
                (r,) = c
                sid = idv[pl.ds(r, 16)][0]
                off = sid - segbase
                e = _find_run_end(idv, sid, r, cnt)

                @pl.when(off >= 0)
                def _():
                    foff = off * 64
                    f0 = slab[pl.ds(foff, 16)]
                    f1 = slab[pl.ds(foff + 16, 16)]
                    f2 = slab[pl.ds(foff + 32, 16)]
                    f3 = slab[pl.ds(foff + 48, 16)]

                    def crow(rr, _):
                        obuf[rr, pl.ds(0, 16)] = ybuf[rr, pl.ds(0, 16)]
                        obuf[rr, pl.ds(16, 16)] = ybuf[rr, pl.ds(16, 16)]
                        obuf[rr, pl.ds(32, 16)] = ybuf[rr, pl.ds(32, 16)]
                        obuf[rr, pl.ds(48, 16)] = ybuf[rr, pl.ds(48, 16)]
                        obuf[rr, pl.ds(64, 16)] = f0
                        obuf[rr, pl.ds(80, 16)] = f1
                        obuf[rr, pl.ds(96, 16)] = f2
                        obuf[rr, pl.ds(112, 16)] = f3
                        return 0

                    lax.fori_loop(r, e, crow, 0)

                @pl.when(off < 0)
                def _():
                    pltpu.sync_copy(feat_hbm.at[pl.ds(sid * 64, 64)], fbuf)
                    f0 = fbuf[pl.ds(0, 16)]
                    f1 = fbuf[pl.ds(16, 16)]
                    f2 = fbuf[pl.ds(32, 16)]
                    f3 = fbuf[pl.ds(48, 16)]

                    def crow(rr, _):
                        obuf[rr, pl.ds(0, 16)] = ybuf[rr, pl.ds(0, 16)]
                        obuf[rr, pl.ds(16, 16)] = ybuf[rr, pl.ds(16, 16)]
                        obuf[rr, pl.ds(32, 16)] = ybuf[rr, pl.ds(32, 16)]
                        obuf[rr, pl.ds(48, 16)] = ybuf[rr, pl.ds(48, 16)]
                        obuf[rr, pl.ds(64, 16)] = f0
                        obuf[rr, pl.ds(80, 16)] = f1
                        obuf[rr, pl.ds(96, 16)] = f2
                        obuf[rr, pl.ds(112, 16)] = f3
                        return 0

                    lax.fori_loop(r, e, crow, 0)

                return (e,)

            lax.while_loop(lambda c: c[0] < cnt, run_body, (jnp.int32(0),))
            _tail_start(obuf, out_hbm, base, cnt, osems[b])

        def outer(cj, _):
            for b in range(2):
                ci = cj * 2 + b

                @pl.when(ci < nch)
                def _(ci=ci, b=b):
                    process(ci, b)

                    @pl.when(ci + 2 < nch)
                    def _():
                        issue(ci + 2, b)

            return 0

        lax.fori_loop(0, (nch + 1) // 2, outer, 0)

        for b in range(2):
            cl = jnp.where((nch - 1) % 2 == b, nch - 1, nch - 2)

            @pl.when(cl >= 0)
            def _(b=b, cl=cl):
                pb = pl.multiple_of(lo + cl * CHUNK2, 8)
                pcnt = jnp.minimum(CHUNK2, n - cl * CHUNK2)
                _tail_wait(obufs[b], out_hbm, pb, pcnt, osems[b])


def _sc_params():
    cp = pltpu.CompilerParams()
    if "needs_layout_passes" in pltpu.CompilerParams.__dataclass_fields__:
        cp = dataclasses.replace(cp, needs_layout_passes=False)
    return cp


def _run_sc(y, ids_pad, bounds, walign):
    mesh = plsc.VectorSubcoreMesh(core_axis_name="c", subcore_axis_name="s")
    cp = _sc_params()
    feat = pl.kernel(
        _sc1_body,
        out_type=jax.ShapeDtypeStruct((NSEG_PAD * 64,), jnp.float32),
        mesh=mesh,
        compiler_params=cp,
        scratch_types=[
            pltpu.VMEM((56,), jnp.int32),
            pltpu.VMEM((CHUNK + 40,), jnp.int32),
            pltpu.VMEM((CHUNK + 40,), jnp.int32),
            pltpu.VMEM((CHUNK + 8, UNITS), jnp.float32),
            pltpu.VMEM((CHUNK + 8, UNITS), jnp.float32),
            pltpu.VMEM((SEG_PW * UNITS,), jnp.float32),
            pltpu.SemaphoreType.DMA,
            pltpu.SemaphoreType.DMA,
            pltpu.SemaphoreType.DMA,
            pltpu.SemaphoreType.DMA,
        ],
    )(y, ids_pad, bounds)
    out = pl.kernel(
        _sc2_body,
        out_type=jax.ShapeDtypeStruct((N, 128), jnp.float32),
        mesh=mesh,
        compiler_params=cp,
        scratch_types=[
            pltpu.VMEM((56,), jnp.int32),
            pltpu.VMEM((CHUNK2 + 40,), jnp.int32),
            pltpu.VMEM((CHUNK2 + 40,), jnp.int32),
            pltpu.VMEM((CHUNK2, UNITS), jnp.float32),
            pltpu.VMEM((CHUNK2, UNITS), jnp.float32),
            pltpu.VMEM((SLAB * UNITS,), jnp.float32),
            pltpu.VMEM((UNITS,), jnp.float32),
            pltpu.VMEM((CHUNK2, 128), jnp.float32),
            pltpu.VMEM((CHUNK2, 128), jnp.float32),
            pltpu.SemaphoreType.DMA,
            pltpu.SemaphoreType.DMA,
            pltpu.SemaphoreType.DMA,
            pltpu.SemaphoreType.DMA,
            pltpu.SemaphoreType.DMA,
            pltpu.SemaphoreType.DMA,
        ],
    )(y, ids_pad, walign, feat)
    return out


def kernel(inputs, unq_inv, W, gamma, beta):
    ids = unq_inv.astype(jnp.int32)
    ids3 = ids.reshape(GRID, 1, TILE)
    x, stats, counts = _run_mm_stats(inputs, ids3, W)
    y = _run_bn_swish(x, stats, gamma.reshape(1, UNITS), beta.reshape(1, UNITS))
    c = counts[:NW, 0]
    zero1 = jnp.zeros((1,), jnp.int32)
    pad7 = jnp.zeros((40 - NW - 1,), jnp.int32)
    bounds = jnp.concatenate([zero1, c, pad7])
    wal = jnp.concatenate(
        [zero1, (c[: NW - 1] // 8) * 8, jnp.full((1,), N, jnp.int32), pad7])
    ids_pad = jnp.pad(ids, (0, CHUNK + 8))
    return _run_sc(y, ids_pad, bounds, wal)


# speedup vs baseline: 2.8227x; 1.0446x over previous
import dataclasses
import functools

import jax
import jax.numpy as jnp
from jax import lax
from jax.experimental import pallas as pl
from jax.experimental.pallas import tpu as pltpu
from jax.experimental.pallas import tpu_sc as plsc

N = 320000
IN_CH = 128
UNITS = 64
NUM_SEG = 10000
EPS = 1e-3

TILE = 1280
GRID = N // TILE
NW = 32
CHUNK = 256
CHUNK2 = 96
SEG_PW = (NUM_SEG + NW - 1) // NW + 4
SLAB = SEG_PW + 3
NSEG_PAD = NUM_SEG + 16
N_PAD = N + CHUNK


def _mm_stats_body(in_ref, ids_ref, w_ref, x_ref, stats_ref, counts_ref,
                   acc_ref):
    i = pl.program_id(0)

    @pl.when(i == 0)
    def _():
        stats_ref[...] = jnp.zeros_like(stats_ref)
        acc_ref[...] = jnp.zeros_like(acc_ref)

    x = lax.dot_general(in_ref[...], w_ref[...],
                        (((1,), (1,)), ((), ())),
                        preferred_element_type=jnp.float32)
    x_ref[...] = x.astype(jnp.bfloat16)
    s = jnp.sum(x, axis=0, keepdims=True)
    s2 = jnp.sum(x * x, axis=0, keepdims=True)
    stats_ref[0:1, :] += s
    stats_ref[1:2, :] += s2

    ids = ids_ref[0, 0, :].astype(jnp.int32)
    wix = lax.broadcasted_iota(jnp.int32, (64, 1), 0)
    thr = ((wix + 1) * NUM_SEG) // NW
    acc_ref[...] += (ids[None, :] < thr).astype(jnp.int32)

    @pl.when(i == GRID - 1)
    def _():
        counts_ref[...] = jnp.sum(acc_ref[...], axis=1, keepdims=True)


def _run_mm_stats(inputs, ids3, W):
    return pl.pallas_call(
        _mm_stats_body,
        grid=(GRID,),
        in_specs=[
            pl.BlockSpec((TILE, IN_CH), lambda i: (i, 0)),
            pl.BlockSpec((1, 1, TILE), lambda i: (i, 0, 0)),
            pl.BlockSpec((UNITS, IN_CH), lambda i: (0, 0)),
        ],
        out_specs=[
            pl.BlockSpec((TILE, UNITS), lambda i: (i, 0)),
            pl.BlockSpec((8, UNITS), lambda i: (0, 0)),
            pl.BlockSpec((64, 1), lambda i: (0, 0)),
        ],
        out_shape=[
            jax.ShapeDtypeStruct((N_PAD, UNITS), jnp.bfloat16),
            jax.ShapeDtypeStruct((8, UNITS), jnp.float32),
            jax.ShapeDtypeStruct((64, 1), jnp.int32),
        ],
        scratch_shapes=[pltpu.VMEM((64, TILE), jnp.int32)],
    )(inputs, ids3, W)


def _bn_swish_body(x_ref, stats_ref, g_ref, b_ref, y_ref):
    mean = stats_ref[0:1, :] / N
    ex2 = stats_ref[1:2, :] / N
    var = ex2 - mean * mean
    inv = lax.rsqrt(var + EPS)
    a = g_ref[...] * inv
    b = b_ref[...] - mean * a
    t = x_ref[...].astype(jnp.float32) * a + b
    y_ref[...] = t * (1.0 / (1.0 + jnp.exp(-t)))


def _run_bn_swish(x, stats, gamma, beta):
    return pl.pallas_call(
        _bn_swish_body,
        grid=(GRID,),
        in_specs=[
            pl.BlockSpec((TILE, UNITS), lambda i: (i, 0)),
            pl.BlockSpec((8, UNITS), lambda i: (0, 0)),
            pl.BlockSpec((1, UNITS), lambda i: (0, 0)),
            pl.BlockSpec((1, UNITS), lambda i: (0, 0)),
        ],
        out_specs=pl.BlockSpec((TILE, UNITS), lambda i: (i, 0)),
        out_shape=jax.ShapeDtypeStruct((N_PAD, UNITS), jnp.float32),
    )(x, stats, gamma, beta)


def _tail_start(src, dst_hbm, base, cnt, sem):
    off = jnp.int32(0)
    sz = 1 << (src.shape[0].bit_length() - 1)
    while sz >= 8:
        here = (cnt & sz) != 0

        @pl.when(here)
        def _(off=off, sz=sz):
            o8 = pl.multiple_of(base + off, 8)
            pltpu.async_copy(src.at[pl.ds(off, sz), :],
                             dst_hbm.at[pl.ds(o8, sz), :], sem)

        off = off + jnp.where(here, sz, 0)
        sz //= 2


def _tail_wait(src, dst_hbm, base, cnt, sem):
    off = jnp.int32(0)
    sz = 1 << (src.shape[0].bit_length() - 1)
    while sz >= 8:
        here = (cnt & sz) != 0

        @pl.when(here)
        def _(off=off, sz=sz):
            o8 = pl.multiple_of(base + off, 8)
            pltpu.make_async_copy(src.at[pl.ds(off, sz), :],
                                  dst_hbm.at[pl.ds(o8, sz), :], sem).wait()

        off = off + jnp.where(here, sz, 0)
        sz //= 2


def _find_run_end(idv, sid, q, qend):
    sidv = jnp.full((16,), sid, jnp.int32)

    def cond(p):
        in_range = p < qend
        same = plsc.all_reduce_ffs(idv[pl.ds(p, 16)] != sidv)[0] >= 16
        return jnp.logical_and(in_range, same)

    p = lax.while_loop(cond, lambda p: p + 16, q)
    f = plsc.all_reduce_ffs(idv[pl.ds(p, 16)] != sidv)[0]
    return jnp.maximum(jnp.minimum(p + f, qend), q + 1)


def _sc1_body(y_hbm, ids_hbm, bounds_hbm, feat_hbm,
              bvbuf, idv0, idv1, ybuf0, ybuf1, feat,
              sem_y0, sem_y1, sem_i0, sem_i1):
    wid = lax.axis_index("c") * 16 + lax.axis_index("s")
    pltpu.sync_copy(bounds_hbm, bvbuf.at[pl.ds(0, 40)])
    lo = bvbuf[pl.ds(wid, 16)][0]
    hi = bvbuf[pl.ds(wid + 1, 16)][0]
    n = hi - lo
    segbase = (wid * NUM_SEG) // NW
    nsegs = ((wid + 1) * NUM_SEG) // NW - segbase
    nch = (n + CHUNK - 1) // CHUNK

    ybufs = (ybuf0, ybuf1)
    idvs = (idv0, idv1)
    sems = ((sem_y0, sem_i0), (sem_y1, sem_i1))

    def issue(ci, b):
        base = lo + ci * CHUNK
        base_al = pl.multiple_of((base // 8) * 8, 8)
        pltpu.async_copy(y_hbm.at[pl.ds(base_al, CHUNK + 8), :],
                         ybufs[b], sems[b][0])
        pltpu.async_copy(ids_hbm.at[pl.ds(base_al, CHUNK + 8)],
                         idvs[b].at[pl.ds(0, CHUNK + 8)], sems[b][1])

    def wait(ci, b):
        base = lo + ci * CHUNK
        base_al = pl.multiple_of((base // 8) * 8, 8)
        pltpu.make_async_copy(y_hbm.at[pl.ds(base_al, CHUNK + 8), :],
                              ybufs[b], sems[b][0]).wait()
        pltpu.make_async_copy(ids_hbm.at[pl.ds(base_al, CHUNK + 8)],
                              idvs[b].at[pl.ds(0, CHUNK + 8)],
                              sems[b][1]).wait()

    minf = jnp.full((16,), -jnp.inf, jnp.float32)

    @pl.when(n > 0)
    def _():
        issue(0, 0)

        @pl.when(nch > 1)
        def _():
            issue(1, 1)

        def process(ci, b, carry):
            ybuf = ybufs[b]
            idv = idvs[b]
            wait(ci, b)
            base = lo + ci * CHUNK
            cnt = jnp.minimum(CHUNK, n - ci * CHUNK)
            o = base - (base // 8) * 8

            def run_body(c):
                r, m0, m1, m2, m3, prev = c
                q = o + r
                sid = idv[pl.ds(q, 16)][0]
                change = sid != prev

                @pl.when(change)
                def _():
                    foff = (prev - segbase) * 64
                    feat[pl.ds(foff, 16)] = m0
                    feat[pl.ds(foff + 16, 16)] = m1
                    feat[pl.ds(foff + 32, 16)] = m2
                    feat[pl.ds(foff + 48, 16)] = m3

                e = _find_run_end(idv, sid, q, o + cnt) - o

                y0 = ybuf[q, pl.ds(0, 16)]
                y1 = ybuf[q, pl.ds(16, 16)]
                y2 = ybuf[q, pl.ds(32, 16)]
                y3 = ybuf[q, pl.ds(48, 16)]
                m0 = jnp.where(change, y0, jnp.maximum(m0, y0))
                m1 = jnp.where(change, y1, jnp.maximum(m1, y1))
                m2 = jnp.where(change, y2, jnp.maximum(m2, y2))
                m3 = jnp.where(change, y3, jnp.maximum(m3, y3))

                def maxrow(rr, mm):
                    m0, m1, m2, m3 = mm
                    qq = o + rr
                    m0 = jnp.maximum(m0, ybuf[qq, pl.ds(0, 16)])
                    m1 = jnp.maximum(m1, ybuf[qq, pl.ds(16, 16)])
                    m2 = jnp.maximum(m2, ybuf[qq, pl.ds(32, 16)])
                    m3 = jnp.maximum(m3, ybuf[qq, pl.ds(48, 16)])
                    return m0, m1, m2, m3

                m0, m1, m2, m3 = lax.fori_loop(r + 1, e, maxrow,
                                               (m0, m1, m2, m3))
                return e, m0, m1, m2, m3, sid

            r, m0, m1, m2, m3, prev = lax.while_loop(
                lambda c: c[0] < cnt, run_body, carry)
            return jnp.int32(0), m0, m1, m2, m3, prev

        def outer(cj, carry):
            c = carry
            for b in range(2):
                ci = cj * 2 + b

                def do(c=c, ci=ci, b=b):
                    c2 = process(ci, b, c)

                    @pl.when(ci + 2 < nch)
                    def _():
                        issue(ci + 2, b)

                    return c2

                c = lax.cond(ci < nch, do, lambda c=c: c)
            return c

        init = (jnp.int32(0), minf, minf, minf, minf,
                segbase.astype(jnp.int32))
        fin = lax.fori_loop(0, (nch + 1) // 2, outer, init)
        _, m0, m1, m2, m3, prev = fin
        foff = (prev - segbase) * 64
        feat[pl.ds(foff, 16)] = m0
        feat[pl.ds(foff + 16, 16)] = m1
        feat[pl.ds(foff + 32, 16)] = m2
        feat[pl.ds(foff + 48, 16)] = m3

        pltpu.sync_copy(feat.at[pl.ds(0, 312 * 64)],
                        feat_hbm.at[pl.ds(segbase * 64, 312 * 64)])

        @pl.when(nsegs > 312)
        def _():
            pltpu.sync_copy(
                feat.at[pl.ds(312 * 64, 64)],
                feat_hbm.at[pl.ds((segbase + 312) * 64, 64)])


def _sc2_body(y_hbm, ids_hbm, wal_hbm, feat_hbm, out_hbm,
              bvbuf, idv0, idv1, ybuf0, ybuf1, slab, fbuf, obuf0, obuf1,
              sem_y0, sem_y1, sem_i0, sem_i1, sem_o0, sem_o1):
    wid = lax.axis_index("c") * 16 + lax.axis_index("s")
    pltpu.sync_copy(wal_hbm, bvbuf.at[pl.ds(0, 40)])
    lo = bvbuf[pl.ds(wid, 16)][0]
    hi = bvbuf[pl.ds(wid + 1, 16)][0]
    n = hi - lo
    segbase = (wid * NUM_SEG) // NW
    nch = (n + CHUNK2 - 1) // CHUNK2

    pltpu.sync_copy(feat_hbm.at[pl.ds(segbase * 64, SLAB * 64)], slab)

    ybufs = (ybuf0, ybuf1)
    idvs = (idv0, idv1)
    obufs = (obuf0, obuf1)
    isems = ((sem_y0, sem_i0), (sem_y1, sem_i1))
    osems = (sem_o0, sem_o1)

    def issue(ci, b):
        base = pl.multiple_of(lo + ci * CHUNK2, 8)
        pltpu.async_copy(y_hbm.at[pl.ds(base, CHUNK2), :],
                         ybufs[b], isems[b][0])
        pltpu.async_copy(ids_hbm.at[pl.ds(base, CHUNK2 + 8)],
                         idvs[b].at[pl.ds(0, CHUNK2 + 8)], isems[b][1])

    def wait_in(ci, b):
        base = pl.multiple_of(lo + ci * CHUNK2, 8)
        pltpu.make_async_copy(y_hbm.at[pl.ds(base, CHUNK2), :],
                              ybufs[b], isems[b][0]).wait()
        pltpu.make_async_copy(ids_hbm.at[pl.ds(base, CHUNK2 + 8)],
                              idvs[b].at[pl.ds(0, CHUNK2 + 8)],
                              isems[b][1]).wait()

    @pl.when(n > 0)
    def _():
        issue(0, 0)

        @pl.when(nch > 1)
        def _():
            issue(1, 1)

        def process(ci, b):
            ybuf = ybufs[b]
            idv = idvs[b]
            obuf = obufs[b]
            wait_in(ci, b)
            base = pl.multiple_of(lo + ci * CHUNK2, 8)
            cnt = jnp.minimum(CHUNK2, n - ci * CHUNK2)

            @pl.when(ci >= 2)
            def _():
                pb = pl.multiple_of(lo + (ci - 2) * CHUNK2, 8)
                pcnt = jnp.minimum(CHUNK2, n - (ci - 2) * CHUNK2)
                _tail_wait(obuf, out_hbm, pb, pcnt, osems[b])

            def run_body(c):
                (r,) = c
                sid = idv[pl.ds(r, 16)][0]
                off = sid - segbase
                e = _find_run_end(idv, sid, r, cnt)

                @pl.when(off >= 0)
                def _():
                    foff = off * 64
                    f0 = slab[pl.ds(foff, 16)]
                    f1 = slab[pl.ds(foff + 16, 16)]
                    f2 = slab[pl.ds(foff + 32, 16)]
                    f3 = slab[pl.ds(foff + 48, 16)]

                    def crow(rr, _):
                        obuf[rr, pl.ds(0, 16)] = ybuf[rr, pl.ds(0, 16)]
                        obuf[rr, pl.ds(16, 16)] = ybuf[rr, pl.ds(16, 16)]
                        obuf[rr, pl.ds(32, 16)] = ybuf[rr, pl.ds(32, 16)]
                        obuf[rr, pl.ds(48, 16)] = ybuf[rr, pl.ds(48, 16)]
                        obuf[rr, pl.ds(64, 16)] = f0
                        obuf[rr, pl.ds(80, 16)] = f1
                        obuf[rr, pl.ds(96, 16)] = f2
                        obuf[rr, pl.ds(112, 16)] = f3
                        return 0

                    lax.fori_loop(r, e, crow, 0)

                @pl.when(off < 0)
                def _():
                    pltpu.sync_copy(feat_hbm.at[pl.ds(sid * 64, 64)], fbuf)
                    f0 = fbuf[pl.ds(0, 16)]
                    f1 = fbuf[pl.ds(16, 16)]
                    f2 = fbuf[pl.ds(32, 16)]
                    f3 = fbuf[pl.ds(48, 16)]

                    def crow(rr, _):
                        obuf[rr, pl.ds(0, 16)] = ybuf[rr, pl.ds(0, 16)]
                        obuf[rr, pl.ds(16, 16)] = ybuf[rr, pl.ds(16, 16)]
                        obuf[rr, pl.ds(32, 16)] = ybuf[rr, pl.ds(32, 16)]
                        obuf[rr, pl.ds(48, 16)] = ybuf[rr, pl.ds(48, 16)]
                        obuf[rr, pl.ds(64, 16)] = f0
                        obuf[rr, pl.ds(80, 16)] = f1
                        obuf[rr, pl.ds(96, 16)] = f2
                        obuf[rr, pl.ds(112, 16)] = f3
                        return 0

                    lax.fori_loop(r, e, crow, 0)

                return (e,)

            lax.while_loop(lambda c: c[0] < cnt, run_body, (jnp.int32(0),))
            _tail_start(obuf, out_hbm, base, cnt, osems[b])

        def outer(cj, _):
            for b in range(2):
                ci = cj * 2 + b

                @pl.when(ci < nch)
                def _(ci=ci, b=b):
                    process(ci, b)

                    @pl.when(ci + 2 < nch)
                    def _():
                        issue(ci + 2, b)

            return 0

        lax.fori_loop(0, (nch + 1) // 2, outer, 0)

        for b in range(2):
            cl = jnp.where((nch - 1) % 2 == b, nch - 1, nch - 2)

            @pl.when(cl >= 0)
            def _(b=b, cl=cl):
                pb = pl.multiple_of(lo + cl * CHUNK2, 8)
                pcnt = jnp.minimum(CHUNK2, n - cl * CHUNK2)
                _tail_wait(obufs[b], out_hbm, pb, pcnt, osems[b])


def _sc_params():
    cp = pltpu.CompilerParams()
    if "needs_layout_passes" in pltpu.CompilerParams.__dataclass_fields__:
        cp = dataclasses.replace(cp, needs_layout_passes=False)
    return cp


def _run_sc(y, ids_pad, bounds, walign):
    mesh = plsc.VectorSubcoreMesh(core_axis_name="c", subcore_axis_name="s")
    cp = _sc_params()
    feat = pl.kernel(
        _sc1_body,
        out_type=jax.ShapeDtypeStruct((NSEG_PAD * 64,), jnp.float32),
        mesh=mesh,
        compiler_params=cp,
        scratch_types=[
            pltpu.VMEM((56,), jnp.int32),
            pltpu.VMEM((CHUNK + 40,), jnp.int32),
            pltpu.VMEM((CHUNK + 40,), jnp.int32),
            pltpu.VMEM((CHUNK + 8, UNITS), jnp.float32),
            pltpu.VMEM((CHUNK + 8, UNITS), jnp.float32),
            pltpu.VMEM((SEG_PW * UNITS,), jnp.float32),
            pltpu.SemaphoreType.DMA,
            pltpu.SemaphoreType.DMA,
            pltpu.SemaphoreType.DMA,
            pltpu.SemaphoreType.DMA,
        ],
    )(y, ids_pad, bounds)
    out = pl.kernel(
        _sc2_body,
        out_type=jax.ShapeDtypeStruct((N, 128), jnp.float32),
        mesh=mesh,
        compiler_params=cp,
        scratch_types=[
            pltpu.VMEM((56,), jnp.int32),
            pltpu.VMEM((CHUNK2 + 40,), jnp.int32),
            pltpu.VMEM((CHUNK2 + 40,), jnp.int32),
            pltpu.VMEM((CHUNK2, UNITS), jnp.float32),
            pltpu.VMEM((CHUNK2, UNITS), jnp.float32),
            pltpu.VMEM((SLAB * UNITS,), jnp.float32),
            pltpu.VMEM((UNITS,), jnp.float32),
            pltpu.VMEM((CHUNK2, 128), jnp.float32),
            pltpu.VMEM((CHUNK2, 128), jnp.float32),
            pltpu.SemaphoreType.DMA,
            pltpu.SemaphoreType.DMA,
            pltpu.SemaphoreType.DMA,
            pltpu.SemaphoreType.DMA,
            pltpu.SemaphoreType.DMA,
            pltpu.SemaphoreType.DMA,
        ],
    )(y, ids_pad, walign, feat)
    return out


def kernel(inputs, unq_inv, W, gamma, beta):
    ids = unq_inv.astype(jnp.int32)
    ids3 = ids.reshape(GRID, 1, TILE)
    x, stats, counts = _run_mm_stats(inputs, ids3, W)
    y = _run_bn_swish(x, stats, gamma.reshape(1, UNITS), beta.reshape(1, UNITS))
    c = counts[:NW, 0]
    zero1 = jnp.zeros((1,), jnp.int32)
    pad7 = jnp.zeros((40 - NW - 1,), jnp.int32)
    bounds = jnp.concatenate([zero1, c, pad7])
    wal = jnp.concatenate(
        [zero1, (c[: NW - 1] // 8) * 8, jnp.full((1,), N, jnp.int32), pad7])
    ids_pad = jnp.pad(ids, (0, CHUNK + 8))
    return _run_sc(y, ids_pad, bounds, wal)


# speedup vs baseline: 3.4442x; 1.2202x over previous
import dataclasses
import functools

import jax
import jax.numpy as jnp
from jax import lax
from jax.experimental import pallas as pl
from jax.experimental.pallas import tpu as pltpu
from jax.experimental.pallas import tpu_sc as plsc

N = 320000
IN_CH = 128
UNITS = 64
NUM_SEG = 10000
EPS = 1e-3

TILE = 2560
GRID = N // TILE
NW = 32
CHUNK = 256
CHUNK2 = 96
SEG_PW = (NUM_SEG + NW - 1) // NW + 4
SLAB = SEG_PW + 3
NSEG_PAD = NUM_SEG + 16
N_PAD = N + CHUNK


def _mm_stats_body(in_ref, ids_ref, w_ref, x_ref, stats_ref, counts_ref,
                   acc_ref):
    i = pl.program_id(0)

    @pl.when(i == 0)
    def _():
        stats_ref[...] = jnp.zeros_like(stats_ref)
        acc_ref[...] = jnp.zeros_like(acc_ref)

    x = lax.dot_general(in_ref[...], w_ref[...],
                        (((1,), (1,)), ((), ())),
                        preferred_element_type=jnp.float32)
    x_ref[...] = x.astype(jnp.bfloat16)
    s = jnp.sum(x, axis=0, keepdims=True)
    s2 = jnp.sum(x * x, axis=0, keepdims=True)
    stats_ref[0:1, :] += s
    stats_ref[1:2, :] += s2

    ids = ids_ref[0, 0, :].astype(jnp.int32)
    wix = lax.broadcasted_iota(jnp.int32, (64, 1), 0)
    thr = ((wix + 1) * NUM_SEG) // NW
    acc_ref[...] += (ids[None, :] < thr).astype(jnp.int32)

    @pl.when(i == GRID - 1)
    def _():
        counts_ref[...] = jnp.sum(acc_ref[...], axis=1, keepdims=True)


def _run_mm_stats(inputs, ids3, W):
    return pl.pallas_call(
        _mm_stats_body,
        grid=(GRID,),
        in_specs=[
            pl.BlockSpec((TILE, IN_CH), lambda i: (i, 0)),
            pl.BlockSpec((1, 1, TILE), lambda i: (i, 0, 0)),
            pl.BlockSpec((UNITS, IN_CH), lambda i: (0, 0)),
        ],
        out_specs=[
            pl.BlockSpec((TILE, UNITS), lambda i: (i, 0)),
            pl.BlockSpec((8, UNITS), lambda i: (0, 0)),
            pl.BlockSpec((64, 1), lambda i: (0, 0)),
        ],
        out_shape=[
            jax.ShapeDtypeStruct((N_PAD, UNITS), jnp.bfloat16),
            jax.ShapeDtypeStruct((8, UNITS), jnp.float32),
            jax.ShapeDtypeStruct((64, 1), jnp.int32),
        ],
        scratch_shapes=[pltpu.VMEM((64, TILE), jnp.int32)],
    )(inputs, ids3, W)


def _bn_swish_body(x_ref, stats_ref, g_ref, b_ref, y_ref):
    mean = stats_ref[0:1, :] / N
    ex2 = stats_ref[1:2, :] / N
    var = ex2 - mean * mean
    inv = lax.rsqrt(var + EPS)
    a = g_ref[...] * inv
    b = b_ref[...] - mean * a
    t = x_ref[...].astype(jnp.float32) * a + b
    y_ref[...] = t * (1.0 / (1.0 + jnp.exp(-t)))


def _run_bn_swish(x, stats, gamma, beta):
    return pl.pallas_call(
        _bn_swish_body,
        grid=(GRID,),
        in_specs=[
            pl.BlockSpec((TILE, UNITS), lambda i: (i, 0)),
            pl.BlockSpec((8, UNITS), lambda i: (0, 0)),
            pl.BlockSpec((1, UNITS), lambda i: (0, 0)),
            pl.BlockSpec((1, UNITS), lambda i: (0, 0)),
        ],
        out_specs=pl.BlockSpec((TILE, UNITS), lambda i: (i, 0)),
        out_shape=jax.ShapeDtypeStruct((N_PAD, UNITS), jnp.float32),
    )(x, stats, gamma, beta)


def _tail_start(src, dst_hbm, base, cnt, sem):
    off = jnp.int32(0)
    sz = 1 << (src.shape[0].bit_length() - 1)
    while sz >= 8:
        here = (cnt & sz) != 0

        @pl.when(here)
        def _(off=off, sz=sz):
            o8 = pl.multiple_of(base + off, 8)
            pltpu.async_copy(src.at[pl.ds(off, sz), :],
                             dst_hbm.at[pl.ds(o8, sz), :], sem)

        off = off + jnp.where(here, sz, 0)
        sz //= 2


def _tail_wait(src, dst_hbm, base, cnt, sem):
    off = jnp.int32(0)
    sz = 1 << (src.shape[0].bit_length() - 1)
    while sz >= 8:
        here = (cnt & sz) != 0

        @pl.when(here)
        def _(off=off, sz=sz):
            o8 = pl.multiple_of(base + off, 8)
            pltpu.make_async_copy(src.at[pl.ds(off, sz), :],
                                  dst_hbm.at[pl.ds(o8, sz), :], sem).wait()

        off = off + jnp.where(here, sz, 0)
        sz //= 2


def _find_run_end(idv, sid, q, qend):
    sidv = jnp.full((16,), sid, jnp.int32)

    def cond(p):
        in_range = p < qend
        same = plsc.all_reduce_ffs(idv[pl.ds(p, 16)] != sidv)[0] >= 16
        return jnp.logical_and(in_range, same)

    p = lax.while_loop(cond, lambda p: p + 16, q)
    f = plsc.all_reduce_ffs(idv[pl.ds(p, 16)] != sidv)[0]
    return jnp.maximum(jnp.minimum(p + f, qend), q + 1)


def _sc1_body(y_hbm, ids_hbm, bounds_hbm, feat_hbm,
              bvbuf, idv0, idv1, ybuf0, ybuf1, feat,
              sem_y0, sem_y1, sem_i0, sem_i1):
    wid = lax.axis_index("c") * 16 + lax.axis_index("s")
    pltpu.sync_copy(bounds_hbm, bvbuf.at[pl.ds(0, 40)])
    lo = bvbuf[pl.ds(wid, 16)][0]
    hi = bvbuf[pl.ds(wid + 1, 16)][0]
    n = hi - lo
    segbase = (wid * NUM_SEG) // NW
    nsegs = ((wid + 1) * NUM_SEG) // NW - segbase
    nch = (n + CHUNK - 1) // CHUNK

    ybufs = (ybuf0, ybuf1)
    idvs = (idv0, idv1)
    sems = ((sem_y0, sem_i0), (sem_y1, sem_i1))

    def issue(ci, b):
        base = lo + ci * CHUNK
        base_al = pl.multiple_of((base // 8) * 8, 8)
        pltpu.async_copy(y_hbm.at[pl.ds(base_al, CHUNK + 8), :],
                         ybufs[b], sems[b][0])
        pltpu.async_copy(ids_hbm.at[pl.ds(base_al, CHUNK + 8)],
                         idvs[b].at[pl.ds(0, CHUNK + 8)], sems[b][1])

    def wait(ci, b):
        base = lo + ci * CHUNK
        base_al = pl.multiple_of((base // 8) * 8, 8)
        pltpu.make_async_copy(y_hbm.at[pl.ds(base_al, CHUNK + 8), :],
                              ybufs[b], sems[b][0]).wait()
        pltpu.make_async_copy(ids_hbm.at[pl.ds(base_al, CHUNK + 8)],
                              idvs[b].at[pl.ds(0, CHUNK + 8)],
                              sems[b][1]).wait()

    minf = jnp.full((16,), -jnp.inf, jnp.float32)

    @pl.when(n > 0)
    def _():
        issue(0, 0)

        @pl.when(nch > 1)
        def _():
            issue(1, 1)

        def process(ci, b, carry):
            ybuf = ybufs[b]
            idv = idvs[b]
            wait(ci, b)
            base = lo + ci * CHUNK
            cnt = jnp.minimum(CHUNK, n - ci * CHUNK)
            o = base - (base // 8) * 8

            def run_body(c):
                r, m0, m1, m2, m3, prev = c
                q = o + r
                sid = idv[pl.ds(q, 16)][0]
                change = sid != prev

                @pl.when(change)
                def _():
                    foff = (prev - segbase) * 64
                    feat[pl.ds(foff, 16)] = m0
                    feat[pl.ds(foff + 16, 16)] = m1
                    feat[pl.ds(foff + 32, 16)] = m2
                    feat[pl.ds(foff + 48, 16)] = m3

                e = _find_run_end(idv, sid, q, o + cnt) - o

                y0 = ybuf[q, pl.ds(0, 16)]
                y1 = ybuf[q, pl.ds(16, 16)]
                y2 = ybuf[q, pl.ds(32, 16)]
                y3 = ybuf[q, pl.ds(48, 16)]
                m0 = jnp.where(change, y0, jnp.maximum(m0, y0))
                m1 = jnp.where(change, y1, jnp.maximum(m1, y1))
                m2 = jnp.where(change, y2, jnp.maximum(m2, y2))
                m3 = jnp.where(change, y3, jnp.maximum(m3, y3))

                def maxrow(rr, mm):
                    m0, m1, m2, m3 = mm
                    qq = o + rr
                    m0 = jnp.maximum(m0, ybuf[qq, pl.ds(0, 16)])
                    m1 = jnp.maximum(m1, ybuf[qq, pl.ds(16, 16)])
                    m2 = jnp.maximum(m2, ybuf[qq, pl.ds(32, 16)])
                    m3 = jnp.maximum(m3, ybuf[qq, pl.ds(48, 16)])
                    return m0, m1, m2, m3

                m0, m1, m2, m3 = lax.fori_loop(r + 1, e, maxrow,
                                               (m0, m1, m2, m3))
                return e, m0, m1, m2, m3, sid

            r, m0, m1, m2, m3, prev = lax.while_loop(
                lambda c: c[0] < cnt, run_body, carry)
            return jnp.int32(0), m0, m1, m2, m3, prev

        def outer(cj, carry):
            c = carry
            for b in range(2):
                ci = cj * 2 + b

                def do(c=c, ci=ci, b=b):
                    c2 = process(ci, b, c)

                    @pl.when(ci + 2 < nch)
                    def _():
                        issue(ci + 2, b)

                    return c2

                c = lax.cond(ci < nch, do, lambda c=c: c)
            return c

        init = (jnp.int32(0), minf, minf, minf, minf,
                segbase.astype(jnp.int32))
        fin = lax.fori_loop(0, (nch + 1) // 2, outer, init)
        _, m0, m1, m2, m3, prev = fin
        foff = (prev - segbase) * 64
        feat[pl.ds(foff, 16)] = m0
        feat[pl.ds(foff + 16, 16)] = m1
        feat[pl.ds(foff + 32, 16)] = m2
        feat[pl.ds(foff + 48, 16)] = m3

        pltpu.sync_copy(feat.at[pl.ds(0, 312 * 64)],
                        feat_hbm.at[pl.ds(segbase * 64, 312 * 64)])

        @pl.when(nsegs > 312)
        def _():
            pltpu.sync_copy(
                feat.at[pl.ds(312 * 64, 64)],
                feat_hbm.at[pl.ds((segbase + 312) * 64, 64)])


def _sc2_body(y_hbm, ids_hbm, wal_hbm, feat_hbm, out_hbm,
              bvbuf, idv0, idv1, ybuf0, ybuf1, slab, fbuf, obuf0, obuf1,
              sem_y0, sem_y1, sem_i0, sem_i1, sem_o0, sem_o1):
    wid = lax.axis_index("c") * 16 + lax.axis_index("s")
    pltpu.sync_copy(wal_hbm, bvbuf.at[pl.ds(0, 40)])
    lo = bvbuf[pl.ds(wid, 16)][0]
    hi = bvbuf[pl.ds(wid + 1, 16)][0]
    n = hi - lo
    segbase = (wid * NUM_SEG) // NW
    nch = (n + CHUNK2 - 1) // CHUNK2

    pltpu.sync_copy(feat_hbm.at[pl.ds(segbase * 64, SLAB * 64)], slab)

    ybufs = (ybuf0, ybuf1)
    idvs = (idv0, idv1)
    obufs = (obuf0, obuf1)
    isems = ((sem_y0, sem_i0), (sem_y1, sem_i1))
    osems = (sem_o0, sem_o1)

    def issue(ci, b):
        base = pl.multiple_of(lo + ci * CHUNK2, 8)
        pltpu.async_copy(y_hbm.at[pl.ds(base, CHUNK2), :],
                         ybufs[b], isems[b][0])
        pltpu.async_copy(ids_hbm.at[pl.ds(base, CHUNK2 + 8)],
                         idvs[b].at[pl.ds(0, CHUNK2 + 8)], isems[b][1])

    def wait_in(ci, b):
        base = pl.multiple_of(lo + ci * CHUNK2, 8)
        pltpu.make_async_copy(y_hbm.at[pl.ds(base, CHUNK2), :],
                              ybufs[b], isems[b][0]).wait()
        pltpu.make_async_copy(ids_hbm.at[pl.ds(base, CHUNK2 + 8)],
                              idvs[b].at[pl.ds(0, CHUNK2 + 8)],
                              isems[b][1]).wait()

    @pl.when(n > 0)
    def _():
        issue(0, 0)

        @pl.when(nch > 1)
        def _():
            issue(1, 1)

        def process(ci, b):
            ybuf = ybufs[b]
            idv = idvs[b]
            obuf = obufs[b]
            wait_in(ci, b)
            base = pl.multiple_of(lo + ci * CHUNK2, 8)
            cnt = jnp.minimum(CHUNK2, n - ci * CHUNK2)

            @pl.when(ci >= 2)
            def _():
                pb = pl.multiple_of(lo + (ci - 2) * CHUNK2, 8)
                pcnt = jnp.minimum(CHUNK2, n - (ci - 2) * CHUNK2)
                _tail_wait(obuf, out_hbm, pb, pcnt, osems[b])

            def run_body(c):
                (r,) = c
                sid = idv[pl.ds(r, 16)][0]
                off = sid - segbase
                e = _find_run_end(idv, sid, r, cnt)

                @pl.when(off >= 0)
                def _():
                    foff = off * 64
                    f0 = slab[pl.ds(foff, 16)]
                    f1 = slab[pl.ds(foff + 16, 16)]
                    f2 = slab[pl.ds(foff + 32, 16)]
                    f3 = slab[pl.ds(foff + 48, 16)]

                    def crow(rr, _):
                        obuf[rr, pl.ds(0, 16)] = ybuf[rr, pl.ds(0, 16)]
                        obuf[rr, pl.ds(16, 16)] = ybuf[rr, pl.ds(16, 16)]
                        obuf[rr, pl.ds(32, 16)] = ybuf[rr, pl.ds(32, 16)]
                        obuf[rr, pl.ds(48, 16)] = ybuf[rr, pl.ds(48, 16)]
                        obuf[rr, pl.ds(64, 16)] = f0
                        obuf[rr, pl.ds(80, 16)] = f1
                        obuf[rr, pl.ds(96, 16)] = f2
                        obuf[rr, pl.ds(112, 16)] = f3
                        return 0

                    lax.fori_loop(r, e, crow, 0)

                @pl.when(off < 0)
                def _():
                    pltpu.sync_copy(feat_hbm.at[pl.ds(sid * 64, 64)], fbuf)
                    f0 = fbuf[pl.ds(0, 16)]
                    f1 = fbuf[pl.ds(16, 16)]
                    f2 = fbuf[pl.ds(32, 16)]
                    f3 = fbuf[pl.ds(48, 16)]

                    def crow(rr, _):
                        obuf[rr, pl.ds(0, 16)] = ybuf[rr, pl.ds(0, 16)]
                        obuf[rr, pl.ds(16, 16)] = ybuf[rr, pl.ds(16, 16)]
                        obuf[rr, pl.ds(32, 16)] = ybuf[rr, pl.ds(32, 16)]
                        obuf[rr, pl.ds(48, 16)] = ybuf[rr, pl.ds(48, 16)]
                        obuf[rr, pl.ds(64, 16)] = f0
                        obuf[rr, pl.ds(80, 16)] = f1
                        obuf[rr, pl.ds(96, 16)] = f2
                        obuf[rr, pl.ds(112, 16)] = f3
                        return 0

                    lax.fori_loop(r, e, crow, 0)

                return (e,)

            lax.while_loop(lambda c: c[0] < cnt, run_body, (jnp.int32(0),))
            _tail_start(obuf, out_hbm, base, cnt, osems[b])

        def outer(cj, _):
            for b in range(2):
                ci = cj * 2 + b

                @pl.when(ci < nch)
                def _(ci=ci, b=b):
                    process(ci, b)

                    @pl.when(ci + 2 < nch)
                    def _():
                        issue(ci + 2, b)

            return 0

        lax.fori_loop(0, (nch + 1) // 2, outer, 0)

        for b in range(2):
            cl = jnp.where((nch - 1) % 2 == b, nch - 1, nch - 2)

            @pl.when(cl >= 0)
            def _(b=b, cl=cl):
                pb = pl.multiple_of(lo + cl * CHUNK2, 8)
                pcnt = jnp.minimum(CHUNK2, n - cl * CHUNK2)
                _tail_wait(obufs[b], out_hbm, pb, pcnt, osems[b])


def _sc_params():
    cp = pltpu.CompilerParams()
    if "needs_layout_passes" in pltpu.CompilerParams.__dataclass_fields__:
        cp = dataclasses.replace(cp, needs_layout_passes=False)
    return cp


def _run_sc(y, ids_pad, bounds, walign):
    mesh = plsc.VectorSubcoreMesh(core_axis_name="c", subcore_axis_name="s")
    cp = _sc_params()
    feat = pl.kernel(
        _sc1_body,
        out_type=jax.ShapeDtypeStruct((NSEG_PAD * 64,), jnp.float32),
        mesh=mesh,
        compiler_params=cp,
        scratch_types=[
            pltpu.VMEM((56,), jnp.int32),
            pltpu.VMEM((CHUNK + 40,), jnp.int32),
            pltpu.VMEM((CHUNK + 40,), jnp.int32),
            pltpu.VMEM((CHUNK + 8, UNITS), jnp.float32),
            pltpu.VMEM((CHUNK + 8, UNITS), jnp.float32),
            pltpu.VMEM((SEG_PW * UNITS,), jnp.float32),
            pltpu.SemaphoreType.DMA,
            pltpu.SemaphoreType.DMA,
            pltpu.SemaphoreType.DMA,
            pltpu.SemaphoreType.DMA,
        ],
    )(y, ids_pad, bounds)
    out = pl.kernel(
        _sc2_body,
        out_type=jax.ShapeDtypeStruct((N, 128), jnp.float32),
        mesh=mesh,
        compiler_params=cp,
        scratch_types=[
            pltpu.VMEM((56,), jnp.int32),
            pltpu.VMEM((CHUNK2 + 40,), jnp.int32),
            pltpu.VMEM((CHUNK2 + 40,), jnp.int32),
            pltpu.VMEM((CHUNK2, UNITS), jnp.float32),
            pltpu.VMEM((CHUNK2, UNITS), jnp.float32),
            pltpu.VMEM((SLAB * UNITS,), jnp.float32),
            pltpu.VMEM((UNITS,), jnp.float32),
            pltpu.VMEM((CHUNK2, 128), jnp.float32),
            pltpu.VMEM((CHUNK2, 128), jnp.float32),
            pltpu.SemaphoreType.DMA,
            pltpu.SemaphoreType.DMA,
            pltpu.SemaphoreType.DMA,
            pltpu.SemaphoreType.DMA,
            pltpu.SemaphoreType.DMA,
            pltpu.SemaphoreType.DMA,
        ],
    )(y, ids_pad, walign, feat)
    return out


def kernel(inputs, unq_inv, W, gamma, beta):
    ids = unq_inv.astype(jnp.int32)
    ids3 = ids.reshape(GRID, 1, TILE)
    x, stats, counts = _run_mm_stats(inputs, ids3, W)
    y = _run_bn_swish(x, stats, gamma.reshape(1, UNITS), beta.reshape(1, UNITS))
    c = counts[:NW, 0]
    zero1 = jnp.zeros((1,), jnp.int32)
    pad7 = jnp.zeros((40 - NW - 1,), jnp.int32)
    bounds = jnp.concatenate([zero1, c, pad7])
    wal = jnp.concatenate(
        [zero1, (c[: NW - 1] // 8) * 8, jnp.full((1,), N, jnp.int32), pad7])
    ids_pad = jnp.pad(ids, (0, CHUNK + 8))
    return _run_sc(y, ids_pad, bounds, wal)


# speedup vs baseline: 4.0408x; 1.1732x over previous
import dataclasses
import functools

import jax
import jax.numpy as jnp
from jax import lax
from jax.experimental import pallas as pl
from jax.experimental.pallas import tpu as pltpu
from jax.experimental.pallas import tpu_sc as plsc

N = 320000
IN_CH = 128
UNITS = 64
NUM_SEG = 10000
EPS = 1e-3

TILE = 6400
GRID = N // TILE
NW = 32
CHUNK = 256
CHUNK2 = 96
SEG_PW = (NUM_SEG + NW - 1) // NW + 4
SLAB = SEG_PW + 3
NSEG_PAD = NUM_SEG + 16
N_PAD = N + CHUNK


def _mm_stats_body(in_ref, ids_ref, w_ref, x_ref, stats_ref, counts_ref,
                   acc_ref):
    i = pl.program_id(0)

    @pl.when(i == 0)
    def _():
        stats_ref[...] = jnp.zeros_like(stats_ref)
        acc_ref[...] = jnp.zeros_like(acc_ref)

    x = lax.dot_general(in_ref[...], w_ref[...],
                        (((1,), (1,)), ((), ())),
                        preferred_element_type=jnp.float32)
    x_ref[...] = x.astype(jnp.bfloat16)
    s = jnp.sum(x, axis=0, keepdims=True)
    s2 = jnp.sum(x * x, axis=0, keepdims=True)
    stats_ref[0:1, :] += s
    stats_ref[1:2, :] += s2

    ids = ids_ref[0, 0, :].astype(jnp.int32)
    wix = lax.broadcasted_iota(jnp.int32, (64, 1), 0)
    thr = ((wix + 1) * NUM_SEG) // NW
    acc_ref[...] += (ids[None, :] < thr).astype(jnp.int32)

    @pl.when(i == GRID - 1)
    def _():
        counts_ref[...] = jnp.sum(acc_ref[...], axis=1, keepdims=True)


def _run_mm_stats(inputs, ids3, W):
    return pl.pallas_call(
        _mm_stats_body,
        grid=(GRID,),
        in_specs=[
            pl.BlockSpec((TILE, IN_CH), lambda i: (i, 0)),
            pl.BlockSpec((1, 1, TILE), lambda i: (i, 0, 0)),
            pl.BlockSpec((UNITS, IN_CH), lambda i: (0, 0)),
        ],
        out_specs=[
            pl.BlockSpec((TILE, UNITS), lambda i: (i, 0)),
            pl.BlockSpec((8, UNITS), lambda i: (0, 0)),
            pl.BlockSpec((64, 1), lambda i: (0, 0)),
        ],
        out_shape=[
            jax.ShapeDtypeStruct((N_PAD, UNITS), jnp.bfloat16),
            jax.ShapeDtypeStruct((8, UNITS), jnp.float32),
            jax.ShapeDtypeStruct((64, 1), jnp.int32),
        ],
        scratch_shapes=[pltpu.VMEM((64, TILE), jnp.int32)],
    )(inputs, ids3, W)


def _bn_swish_body(x_ref, stats_ref, g_ref, b_ref, y_ref):
    mean = stats_ref[0:1, :] / N
    ex2 = stats_ref[1:2, :] / N
    var = ex2 - mean * mean
    inv = lax.rsqrt(var + EPS)
    a = g_ref[...] * inv
    b = b_ref[...] - mean * a
    t = x_ref[...].astype(jnp.float32) * a + b
    y_ref[...] = t * (1.0 / (1.0 + jnp.exp(-t)))


def _run_bn_swish(x, stats, gamma, beta):
    return pl.pallas_call(
        _bn_swish_body,
        grid=(GRID,),
        in_specs=[
            pl.BlockSpec((TILE, UNITS), lambda i: (i, 0)),
            pl.BlockSpec((8, UNITS), lambda i: (0, 0)),
            pl.BlockSpec((1, UNITS), lambda i: (0, 0)),
            pl.BlockSpec((1, UNITS), lambda i: (0, 0)),
        ],
        out_specs=pl.BlockSpec((TILE, UNITS), lambda i: (i, 0)),
        out_shape=jax.ShapeDtypeStruct((N_PAD, UNITS), jnp.float32),
    )(x, stats, gamma, beta)


def _tail_start(src, dst_hbm, base, cnt, sem):
    off = jnp.int32(0)
    sz = 1 << (src.shape[0].bit_length() - 1)
    while sz >= 8:
        here = (cnt & sz) != 0

        @pl.when(here)
        def _(off=off, sz=sz):
            o8 = pl.multiple_of(base + off, 8)
            pltpu.async_copy(src.at[pl.ds(off, sz), :],
                             dst_hbm.at[pl.ds(o8, sz), :], sem)

        off = off + jnp.where(here, sz, 0)
        sz //= 2


def _tail_wait(src, dst_hbm, base, cnt, sem):
    off = jnp.int32(0)
    sz = 1 << (src.shape[0].bit_length() - 1)
    while sz >= 8:
        here = (cnt & sz) != 0

        @pl.when(here)
        def _(off=off, sz=sz):
            o8 = pl.multiple_of(base + off, 8)
            pltpu.make_async_copy(src.at[pl.ds(off, sz), :],
                                  dst_hbm.at[pl.ds(o8, sz), :], sem).wait()

        off = off + jnp.where(here, sz, 0)
        sz //= 2


def _find_run_end(idv, sid, q, qend):
    sidv = jnp.full((16,), sid, jnp.int32)

    def cond(p):
        in_range = p < qend
        same = plsc.all_reduce_ffs(idv[pl.ds(p, 16)] != sidv)[0] >= 16
        return jnp.logical_and(in_range, same)

    p = lax.while_loop(cond, lambda p: p + 16, q)
    f = plsc.all_reduce_ffs(idv[pl.ds(p, 16)] != sidv)[0]
    return jnp.maximum(jnp.minimum(p + f, qend), q + 1)


def _sc1_body(y_hbm, ids_hbm, bounds_hbm, feat_hbm,
              bvbuf, idv0, idv1, ybuf0, ybuf1, feat,
              sem_y0, sem_y1, sem_i0, sem_i1):
    wid = lax.axis_index("c") * 16 + lax.axis_index("s")
    pltpu.sync_copy(bounds_hbm, bvbuf.at[pl.ds(0, 40)])
    lo = bvbuf[pl.ds(wid, 16)][0]
    hi = bvbuf[pl.ds(wid + 1, 16)][0]
    n = hi - lo
    segbase = (wid * NUM_SEG) // NW
    nsegs = ((wid + 1) * NUM_SEG) // NW - segbase
    nch = (n + CHUNK - 1) // CHUNK

    ybufs = (ybuf0, ybuf1)
    idvs = (idv0, idv1)
    sems = ((sem_y0, sem_i0), (sem_y1, sem_i1))

    def issue(ci, b):
        base = lo + ci * CHUNK
        base_al = pl.multiple_of((base // 8) * 8, 8)
        pltpu.async_copy(y_hbm.at[pl.ds(base_al, CHUNK + 8), :],
                         ybufs[b], sems[b][0])
        pltpu.async_copy(ids_hbm.at[pl.ds(base_al, CHUNK + 8)],
                         idvs[b].at[pl.ds(0, CHUNK + 8)], sems[b][1])

    def wait(ci, b):
        base = lo + ci * CHUNK
        base_al = pl.multiple_of((base // 8) * 8, 8)
        pltpu.make_async_copy(y_hbm.at[pl.ds(base_al, CHUNK + 8), :],
                              ybufs[b], sems[b][0]).wait()
        pltpu.make_async_copy(ids_hbm.at[pl.ds(base_al, CHUNK + 8)],
                              idvs[b].at[pl.ds(0, CHUNK + 8)],
                              sems[b][1]).wait()

    minf = jnp.full((16,), -jnp.inf, jnp.float32)

    @pl.when(n > 0)
    def _():
        issue(0, 0)

        @pl.when(nch > 1)
        def _():
            issue(1, 1)

        def process(ci, b, carry):
            ybuf = ybufs[b]
            idv = idvs[b]
            wait(ci, b)
            base = lo + ci * CHUNK
            cnt = jnp.minimum(CHUNK, n - ci * CHUNK)
            o = base - (base // 8) * 8

            def run_body(c):
                r, m0, m1, m2, m3, prev = c
                q = o + r
                sid = idv[pl.ds(q, 16)][0]
                change = sid != prev

                @pl.when(change)
                def _():
                    foff = (prev - segbase) * 64
                    feat[pl.ds(foff, 16)] = m0
                    feat[pl.ds(foff + 16, 16)] = m1
                    feat[pl.ds(foff + 32, 16)] = m2
                    feat[pl.ds(foff + 48, 16)] = m3

                e = _find_run_end(idv, sid, q, o + cnt) - o

                y0 = ybuf[q, pl.ds(0, 16)]
                y1 = ybuf[q, pl.ds(16, 16)]
                y2 = ybuf[q, pl.ds(32, 16)]
                y3 = ybuf[q, pl.ds(48, 16)]
                m0 = jnp.where(change, y0, jnp.maximum(m0, y0))
                m1 = jnp.where(change, y1, jnp.maximum(m1, y1))
                m2 = jnp.where(change, y2, jnp.maximum(m2, y2))
                m3 = jnp.where(change, y3, jnp.maximum(m3, y3))

                def maxrow(rr, mm):
                    m0, m1, m2, m3 = mm
                    qq = o + rr
                    m0 = jnp.maximum(m0, ybuf[qq, pl.ds(0, 16)])
                    m1 = jnp.maximum(m1, ybuf[qq, pl.ds(16, 16)])
                    m2 = jnp.maximum(m2, ybuf[qq, pl.ds(32, 16)])
                    m3 = jnp.maximum(m3, ybuf[qq, pl.ds(48, 16)])
                    return m0, m1, m2, m3

                m0, m1, m2, m3 = lax.fori_loop(r + 1, e, maxrow,
                                               (m0, m1, m2, m3))
                return e, m0, m1, m2, m3, sid

            r, m0, m1, m2, m3, prev = lax.while_loop(
                lambda c: c[0] < cnt, run_body, carry)
            return jnp.int32(0), m0, m1, m2, m3, prev

        def outer(cj, carry):
            c = carry
            for b in range(2):
                ci = cj * 2 + b

                def do(c=c, ci=ci, b=b):
                    c2 = process(ci, b, c)

                    @pl.when(ci + 2 < nch)
                    def _():
                        issue(ci + 2, b)

                    return c2

                c = lax.cond(ci < nch, do, lambda c=c: c)
            return c

        init = (jnp.int32(0), minf, minf, minf, minf,
                segbase.astype(jnp.int32))
        fin = lax.fori_loop(0, (nch + 1) // 2, outer, init)
        _, m0, m1, m2, m3, prev = fin
        foff = (prev - segbase) * 64
        feat[pl.ds(foff, 16)] = m0
        feat[pl.ds(foff + 16, 16)] = m1
        feat[pl.ds(foff + 32, 16)] = m2
        feat[pl.ds(foff + 48, 16)] = m3

        pltpu.sync_copy(feat.at[pl.ds(0, 312 * 64)],
                        feat_hbm.at[pl.ds(segbase * 64, 312 * 64)])

        @pl.when(nsegs > 312)
        def _():
            pltpu.sync_copy(
                feat.at[pl.ds(312 * 64, 64)],
                feat_hbm.at[pl.ds((segbase + 312) * 64, 64)])


def _sc2_body(y_hbm, ids_hbm, wal_hbm, feat_hbm, out_hbm,
              bvbuf, idv0, idv1, ybuf0, ybuf1, slab, fbuf, obuf0, obuf1,
              sem_y0, sem_y1, sem_i0, sem_i1, sem_o0, sem_o1):
    wid = lax.axis_index("c") * 16 + lax.axis_index("s")
    pltpu.sync_copy(wal_hbm, bvbuf.at[pl.ds(0, 40)])
    lo = bvbuf[pl.ds(wid, 16)][0]
    hi = bvbuf[pl.ds(wid + 1, 16)][0]
    n = hi - lo
    segbase = (wid * NUM_SEG) // NW
    nch = (n + CHUNK2 - 1) // CHUNK2

    pltpu.sync_copy(feat_hbm.at[pl.ds(segbase * 64, SLAB * 64)], slab)

    ybufs = (ybuf0, ybuf1)
    idvs = (idv0, idv1)
    obufs = (obuf0, obuf1)
    isems = ((sem_y0, sem_i0), (sem_y1, sem_i1))
    osems = (sem_o0, sem_o1)

    def issue(ci, b):
        base = pl.multiple_of(lo + ci * CHUNK2, 8)
        pltpu.async_copy(y_hbm.at[pl.ds(base, CHUNK2), :],
                         ybufs[b], isems[b][0])
        pltpu.async_copy(ids_hbm.at[pl.ds(base, CHUNK2 + 8)],
                         idvs[b].at[pl.ds(0, CHUNK2 + 8)], isems[b][1])

    def wait_in(ci, b):
        base = pl.multiple_of(lo + ci * CHUNK2, 8)
        pltpu.make_async_copy(y_hbm.at[pl.ds(base, CHUNK2), :],
                              ybufs[b], isems[b][0]).wait()
        pltpu.make_async_copy(ids_hbm.at[pl.ds(base, CHUNK2 + 8)],
                              idvs[b].at[pl.ds(0, CHUNK2 + 8)],
                              isems[b][1]).wait()

    @pl.when(n > 0)
    def _():
        issue(0, 0)

        @pl.when(nch > 1)
        def _():
            issue(1, 1)

        def process(ci, b):
            ybuf = ybufs[b]
            idv = idvs[b]
            obuf = obufs[b]
            wait_in(ci, b)
            base = pl.multiple_of(lo + ci * CHUNK2, 8)
            cnt = jnp.minimum(CHUNK2, n - ci * CHUNK2)

            @pl.when(ci >= 2)
            def _():
                pb = pl.multiple_of(lo + (ci - 2) * CHUNK2, 8)
                pcnt = jnp.minimum(CHUNK2, n - (ci - 2) * CHUNK2)
                _tail_wait(obuf, out_hbm, pb, pcnt, osems[b])

            def run_body(c):
                (r,) = c
                sid = idv[pl.ds(r, 16)][0]
                off = sid - segbase
                e = _find_run_end(idv, sid, r, cnt)

                @pl.when(off >= 0)
                def _():
                    foff = off * 64
                    f0 = slab[pl.ds(foff, 16)]
                    f1 = slab[pl.ds(foff + 16, 16)]
                    f2 = slab[pl.ds(foff + 32, 16)]
                    f3 = slab[pl.ds(foff + 48, 16)]

                    def crow(rr, _):
                        obuf[rr, pl.ds(0, 16)] = ybuf[rr, pl.ds(0, 16)]
                        obuf[rr, pl.ds(16, 16)] = ybuf[rr, pl.ds(16, 16)]
                        obuf[rr, pl.ds(32, 16)] = ybuf[rr, pl.ds(32, 16)]
                        obuf[rr, pl.ds(48, 16)] = ybuf[rr, pl.ds(48, 16)]
                        obuf[rr, pl.ds(64, 16)] = f0
                        obuf[rr, pl.ds(80, 16)] = f1
                        obuf[rr, pl.ds(96, 16)] = f2
                        obuf[rr, pl.ds(112, 16)] = f3
                        return 0

                    lax.fori_loop(r, e, crow, 0)

                @pl.when(off < 0)
                def _():
                    pltpu.sync_copy(feat_hbm.at[pl.ds(sid * 64, 64)], fbuf)
                    f0 = fbuf[pl.ds(0, 16)]
                    f1 = fbuf[pl.ds(16, 16)]
                    f2 = fbuf[pl.ds(32, 16)]
                    f3 = fbuf[pl.ds(48, 16)]

                    def crow(rr, _):
                        obuf[rr, pl.ds(0, 16)] = ybuf[rr, pl.ds(0, 16)]
                        obuf[rr, pl.ds(16, 16)] = ybuf[rr, pl.ds(16, 16)]
                        obuf[rr, pl.ds(32, 16)] = ybuf[rr, pl.ds(32, 16)]
                        obuf[rr, pl.ds(48, 16)] = ybuf[rr, pl.ds(48, 16)]
                        obuf[rr, pl.ds(64, 16)] = f0
                        obuf[rr, pl.ds(80, 16)] = f1
                        obuf[rr, pl.ds(96, 16)] = f2
                        obuf[rr, pl.ds(112, 16)] = f3
                        return 0

                    lax.fori_loop(r, e, crow, 0)

                return (e,)

            lax.while_loop(lambda c: c[0] < cnt, run_body, (jnp.int32(0),))
            _tail_start(obuf, out_hbm, base, cnt, osems[b])

        def outer(cj, _):
            for b in range(2):
                ci = cj * 2 + b

                @pl.when(ci < nch)
                def _(ci=ci, b=b):
                    process(ci, b)

                    @pl.when(ci + 2 < nch)
                    def _():
                        issue(ci + 2, b)

            return 0

        lax.fori_loop(0, (nch + 1) // 2, outer, 0)

        for b in range(2):
            cl = jnp.where((nch - 1) % 2 == b, nch - 1, nch - 2)

            @pl.when(cl >= 0)
            def _(b=b, cl=cl):
                pb = pl.multiple_of(lo + cl * CHUNK2, 8)
                pcnt = jnp.minimum(CHUNK2, n - cl * CHUNK2)
                _tail_wait(obufs[b], out_hbm, pb, pcnt, osems[b])


def _sc_params():
    cp = pltpu.CompilerParams()
    if "needs_layout_passes" in pltpu.CompilerParams.__dataclass_fields__:
        cp = dataclasses.replace(cp, needs_layout_passes=False)
    return cp


def _run_sc(y, ids_pad, bounds, walign):
    mesh = plsc.VectorSubcoreMesh(core_axis_name="c", subcore_axis_name="s")
    cp = _sc_params()
    feat = pl.kernel(
        _sc1_body,
        out_type=jax.ShapeDtypeStruct((NSEG_PAD * 64,), jnp.float32),
        mesh=mesh,
        compiler_params=cp,
        scratch_types=[
            pltpu.VMEM((56,), jnp.int32),
            pltpu.VMEM((CHUNK + 40,), jnp.int32),
            pltpu.VMEM((CHUNK + 40,), jnp.int32),
            pltpu.VMEM((CHUNK + 8, UNITS), jnp.float32),
            pltpu.VMEM((CHUNK + 8, UNITS), jnp.float32),
            pltpu.VMEM((SEG_PW * UNITS,), jnp.float32),
            pltpu.SemaphoreType.DMA,
            pltpu.SemaphoreType.DMA,
            pltpu.SemaphoreType.DMA,
            pltpu.SemaphoreType.DMA,
        ],
    )(y, ids_pad, bounds)
    out = pl.kernel(
        _sc2_body,
        out_type=jax.ShapeDtypeStruct((N, 128), jnp.float32),
        mesh=mesh,
        compiler_params=cp,
        scratch_types=[
            pltpu.VMEM((56,), jnp.int32),
            pltpu.VMEM((CHUNK2 + 40,), jnp.int32),
            pltpu.VMEM((CHUNK2 + 40,), jnp.int32),
            pltpu.VMEM((CHUNK2, UNITS), jnp.float32),
            pltpu.VMEM((CHUNK2, UNITS), jnp.float32),
            pltpu.VMEM((SLAB * UNITS,), jnp.float32),
            pltpu.VMEM((UNITS,), jnp.float32),
            pltpu.VMEM((CHUNK2, 128), jnp.float32),
            pltpu.VMEM((CHUNK2, 128), jnp.float32),
            pltpu.SemaphoreType.DMA,
            pltpu.SemaphoreType.DMA,
            pltpu.SemaphoreType.DMA,
            pltpu.SemaphoreType.DMA,
            pltpu.SemaphoreType.DMA,
            pltpu.SemaphoreType.DMA,
        ],
    )(y, ids_pad, walign, feat)
    return out


def kernel(inputs, unq_inv, W, gamma, beta):
    ids = unq_inv.astype(jnp.int32)
    ids3 = ids.reshape(GRID, 1, TILE)
    x, stats, counts = _run_mm_stats(inputs, ids3, W)
    y = _run_bn_swish(x, stats, gamma.reshape(1, UNITS), beta.reshape(1, UNITS))
    c = counts[:NW, 0]
    zero1 = jnp.zeros((1,), jnp.int32)
    pad7 = jnp.zeros((40 - NW - 1,), jnp.int32)
    bounds = jnp.concatenate([zero1, c, pad7])
    wal = jnp.concatenate(
        [zero1, (c[: NW - 1] // 8) * 8, jnp.full((1,), N, jnp.int32), pad7])
    ids_pad = jnp.pad(ids, (0, CHUNK + 8))
    return _run_sc(y, ids_pad, bounds, wal)
